# Initial kernel scaffold; baseline (speedup 1.0000x reference)
import jax
import jax.numpy as jnp
from jax.experimental import pallas as pl


def kernel(r, e, a):
    raise NotImplementedError("write your pallas kernel here")


# speedup vs baseline: 2.2958x; 2.2958x over previous
import functools

import jax
import jax.numpy as jnp
from jax import lax
from jax.experimental import pallas as pl
from jax.experimental.pallas import tpu as pltpu
from jax.experimental.pallas import tpu_sc as plsc

D = 128
L = 16
NC = 2
NS = 16
NW = NC * NS
CH = 80
CE = CH // 2
NB = 2


def _make_sc_call(E):
    idx_per_w = 2 * E // NW
    gpw = idx_per_w // CH
    epw = E // NW
    assert idx_per_w * NW == 2 * E and gpw * CH == idx_per_w and NB * (gpw // NB) == gpw

    mesh = plsc.VectorSubcoreMesh(
        core_axis_name="c", subcore_axis_name="s", num_cores=NC, num_subcores=NS
    )

    @functools.partial(
        pl.kernel,
        mesh=mesh,
        out_type=jax.ShapeDtypeStruct((E, D), jnp.float32),
        scratch_types=[
            pltpu.VMEM((gpw, CH), jnp.int32),
            pltpu.VMEM((CH, D), jnp.float32),
            pltpu.VMEM((CH, D), jnp.float32),
            pltpu.VMEM((CE, D), jnp.float32),
            pltpu.VMEM((CE, D), jnp.float32),
            pltpu.SemaphoreType.DMA,
            pltpu.SemaphoreType.DMA,
            pltpu.SemaphoreType.DMA,
            pltpu.SemaphoreType.DMA,
        ],
    )
    def sc_call(r_hbm, idx_hbm, out_hbm,
                idx_v, rows0, rows1, sum0, sum1, gs0, gs1, os0, os1):
        wid = lax.axis_index("s") * NC + lax.axis_index("c")
        pltpu.sync_copy(idx_hbm.at[wid], idx_v)

        rows = (rows0, rows1)
        sums = (sum0, sum1)
        gsems = (gs0, gs1)
        osems = (os0, os1)
        ebase = wid * epw

        for b in range(NB):
            pltpu.async_copy(r_hbm.at[idx_v.at[b]], rows[b], gsems[b])

        def outer(k, carry):
            not_first = k > 0
            not_last = k < gpw // NB - 1
            for b in range(NB):
                g = k * NB + b
                pltpu.make_async_copy(r_hbm.at[idx_v.at[g]], rows[b], gsems[b]).wait()

                @pl.when(not_first)
                def _():
                    pltpu.make_async_copy(
                        sums[b], out_hbm.at[pl.ds(0, CE)], osems[b]
                    ).wait()

                def pair_sum(i, c):
                    for j in range(D // L):
                        sl = pl.ds(j * L, L)
                        sums[b][i, sl] = rows[b][2 * i, sl] + rows[b][2 * i + 1, sl]
                    return c

                lax.fori_loop(0, CE, pair_sum, 0)

                @pl.when(not_last)
                def _():
                    pltpu.async_copy(r_hbm.at[idx_v.at[g + NB]], rows[b], gsems[b])

                pltpu.async_copy(
                    sums[b], out_hbm.at[pl.ds(ebase + g * CE, CE)], osems[b]
                )
            return carry

        lax.fori_loop(0, gpw // NB, outer, 0)
        for b in range(NB):
            pltpu.make_async_copy(sums[b], out_hbm.at[pl.ds(0, CE)], osems[b]).wait()

    return sc_call


def kernel(r, e, a):
    del e
    E = a.shape[0]
    idx = a.astype(jnp.int32).reshape(NW, 2 * E // (NW * CH), CH)
    return _make_sc_call(E)(r.astype(jnp.float32), idx)


# speedup vs baseline: 4.4484x; 1.9376x over previous
import functools

import jax
import jax.numpy as jnp
from jax import lax
from jax.experimental import pallas as pl
from jax.experimental.pallas import tpu as pltpu
from jax.experimental.pallas import tpu_sc as plsc

D = 128
L = 16
NC = 2
NS = 16
NW = NC * NS
CH = 80
CE = CH // 2
NB = 2
SC_G = 50


def _make_sc_call(N, E):
    idx_per_w = 2 * E // NW
    gpw = idx_per_w // CH
    epw = E // NW
    scn = gpw // SC_G
    ki = gpw // NB
    kper = SC_G // NB
    assert idx_per_w * NW == 2 * E and gpw * CH == idx_per_w
    assert NB * ki == gpw and scn * SC_G == gpw and kper * NB == SC_G

    mesh = plsc.VectorSubcoreMesh(
        core_axis_name="c", subcore_axis_name="s", num_cores=NC, num_subcores=NS
    )

    @functools.partial(
        pl.kernel,
        mesh=mesh,
        out_type=jax.ShapeDtypeStruct((E, D), jnp.float32),
        scratch_types=[
            pltpu.VMEM_SHARED((N, D), jnp.float32),
            pltpu.VMEM((2, SC_G, CH), jnp.int32),
            pltpu.VMEM((CH, D), jnp.float32),
            pltpu.VMEM((CH, D), jnp.float32),
            pltpu.VMEM((CE, D), jnp.float32),
            pltpu.VMEM((CE, D), jnp.float32),
            pltpu.SemaphoreType.DMA,
            pltpu.SemaphoreType.DMA,
            pltpu.SemaphoreType.DMA,
            pltpu.SemaphoreType.DMA,
            pltpu.SemaphoreType.DMA,
        ],
    )
    def sc_call(r_hbm, idx_hbm, out_hbm,
                table, idx_v, rows0, rows1, sum0, sum1, gs0, gs1, os0, os1, isem):
        sid = lax.axis_index("s")
        wid = sid * NC + lax.axis_index("c")

        @pl.when(sid == 0)
        def _():
            pltpu.sync_copy(r_hbm, table)

        pltpu.sync_copy(idx_hbm.at[wid, 0], idx_v.at[0])
        plsc.subcore_barrier()

        rows = (rows0, rows1)
        sums = (sum0, sum1)
        gsems = (gs0, gs1)
        osems = (os0, os1)
        ebase = wid * epw

        for b in range(NB):
            pltpu.async_copy(table.at[idx_v.at[0, b]], rows[b], gsems[b])

        def outer(k, carry):
            not_first = k > 0
            not_last = k < ki - 1
            kmod = lax.rem(k, kper)
            has_next_chunk = k < kper * (scn - 1)

            @pl.when((kmod == 0) & has_next_chunk)
            def _():
                mm = k // kper + 1
                pltpu.async_copy(
                    idx_hbm.at[wid, mm], idx_v.at[lax.rem(mm, 2)], isem
                )

            @pl.when((kmod == kper - 1) & has_next_chunk)
            def _():
                pltpu.make_async_copy(
                    idx_hbm.at[wid, 0], idx_v.at[0], isem
                ).wait()

            for b in range(NB):
                g = k * NB + b
                pltpu.make_async_copy(
                    table.at[idx_v.at[0, 0]], rows[b], gsems[b]
                ).wait()

                @pl.when(not_first)
                def _():
                    pltpu.make_async_copy(
                        sums[b], out_hbm.at[pl.ds(0, CE)], osems[b]
                    ).wait()

                @plsc.parallel_loop(0, CE, unroll=4)
                def _(i):
                    for j in range(D // L):
                        sl = pl.ds(j * L, L)
                        sums[b][i, sl] = rows[b][2 * i, sl] + rows[b][2 * i + 1, sl]

                @pl.when(not_last)
                def _():
                    gn = g + NB
                    hn = lax.rem(gn // SC_G, 2)
                    ggn = lax.rem(gn, SC_G)
                    pltpu.async_copy(table.at[idx_v.at[hn, ggn]], rows[b], gsems[b])

                pltpu.async_copy(
                    sums[b], out_hbm.at[pl.ds(ebase + g * CE, CE)], osems[b]
                )
            return carry

        lax.fori_loop(0, ki, outer, 0)
        for b in range(NB):
            pltpu.make_async_copy(sums[b], out_hbm.at[pl.ds(0, CE)], osems[b]).wait()

    return sc_call


def kernel(r, e, a):
    del e
    E = a.shape[0]
    gpw = 2 * E // (NW * CH)
    idx = a.astype(jnp.int32).reshape(NW, gpw // SC_G, SC_G, CH)
    return _make_sc_call(r.shape[0], E)(r.astype(jnp.float32), idx)
